# baseline (speedup 1.0000x reference)
import functools

import jax
import jax.numpy as jnp
from jax import lax
from jax.experimental import pallas as pl
from jax.experimental.pallas import tpu as pltpu
from jax.experimental.pallas import tpu_sc as plsc

NC, NS, L = 2, 16, 16
NW = NC * NS
CH = 80
NB = 4


def _xw_body(coeff_ref, bases_ref, loop_ref, x_ref, out_ref):
    r = pl.program_id(0)
    R, B = coeff_ref.shape
    rc = jnp.minimum(r, R - 1)
    W = coeff_ref[rc, 0] * bases_ref[0]
    for b in range(1, B):
        W = W + coeff_ref[rc, b] * bases_ref[b]
    W = jnp.where(r < R, W, loop_ref[...])
    out_ref[0] = jnp.dot(x_ref[...], W, preferred_element_type=jnp.float32)


def _make_xw(N, IN, OUT, R, B, blkn):
    grid = (R + 1, N // blkn)
    return pl.pallas_call(
        _xw_body,
        grid=grid,
        in_specs=[
            pl.BlockSpec(memory_space=pltpu.SMEM),
            pl.BlockSpec((B, IN, OUT), lambda r, i: (0, 0, 0)),
            pl.BlockSpec((IN, OUT), lambda r, i: (0, 0)),
            pl.BlockSpec((blkn, IN), lambda r, i: (i, 0)),
        ],
        out_specs=pl.BlockSpec((1, blkn, OUT), lambda r, i: (r, i, 0)),
        out_shape=jax.ShapeDtypeStruct((R + 1, N, OUT), jnp.float32),
    )


def _make_deg(N, E, RN):
    epw = E // NW
    mesh = plsc.VectorSubcoreMesh(core_axis_name="c", subcore_axis_name="s",
                                  num_cores=NC, num_subcores=NS)

    @functools.partial(
        pl.kernel,
        out_type=jax.ShapeDtypeStruct((NW, RN), jnp.float32),
        mesh=mesh,
        compiler_params=pltpu.CompilerParams(needs_layout_passes=False),
        scratch_types=[
            pltpu.VMEM((epw,), jnp.int32),
            pltpu.VMEM((epw,), jnp.int32),
            pltpu.VMEM((RN,), jnp.float32),
        ],
    )
    def deg_kernel(pk_hbm, zeros_hbm, out_hbm, typ_v, dst_v, deg_v):
        c = lax.axis_index("c")
        s = lax.axis_index("s")
        wid = s * NC + c
        ebase = wid * epw
        pltpu.sync_copy(pk_hbm.at[pl.ds(2 * E + ebase, epw)], typ_v)
        pltpu.sync_copy(pk_hbm.at[pl.ds(E + ebase, epw)], dst_v)
        pltpu.sync_copy(zeros_hbm, deg_v)
        ones = jnp.ones((L,), jnp.float32)
        gpi = 5

        def step(i, carry):
            for g in range(gpi):
                sl = pl.ds(i * (gpi * L) + g * L, L)
                idx = typ_v[sl] * N + dst_v[sl]
                plsc.addupdate_scatter(deg_v, [idx], ones)
            return carry

        lax.fori_loop(0, epw // (gpi * L), step, 0)
        pltpu.sync_copy(deg_v, out_hbm.at[wid])

    return deg_kernel


def _dinv_body(degp_ref, out_ref):
    total = jnp.sum(degp_ref[...], axis=0)
    out_ref[...] = 1.0 / jnp.maximum(total, 1.0)


def _make_dinv(RN):
    return pl.pallas_call(
        _dinv_body,
        out_shape=jax.ShapeDtypeStruct((RN,), jnp.float32),
    )


def _make_scatter(N, E, OUT, RN):
    epw = E // NW
    nch = epw // CH
    ncopy = 10
    rps = N // ncopy
    niter = nch + 3
    assert niter % NB == 0
    mesh = plsc.VectorSubcoreMesh(core_axis_name="c", subcore_axis_name="s",
                                  num_cores=NC, num_subcores=NS)

    @functools.partial(
        pl.kernel,
        out_type=jax.ShapeDtypeStruct((NC, N, OUT), jnp.float32),
        mesh=mesh,
        compiler_params=pltpu.CompilerParams(needs_layout_passes=False),
        scratch_types=[
            [pltpu.VMEM((CH,), jnp.int32)] * NB,
            [pltpu.VMEM((CH,), jnp.int32)] * NB,
            [pltpu.VMEM((CH,), jnp.int32)] * NB,
            [pltpu.VMEM((CH,), jnp.int32)] * NB,
            [pltpu.VMEM((CH,), jnp.int32)] * NB,
            [pltpu.VMEM((CH,), jnp.float32)] * NB,
            [pltpu.VMEM((CH,), jnp.int32)] * NB,
            [pltpu.VMEM((CH, OUT), jnp.float32)] * NB,
            [pltpu.SemaphoreType.DMA] * NB,
            [pltpu.SemaphoreType.DMA] * NB,
            [pltpu.SemaphoreType.DMA] * NB,
            [pltpu.SemaphoreType.DMA] * NB,
            pltpu.VMEM_SHARED((N, OUT), jnp.float32),
        ],
    )
    def scatter_kernel(pk_hbm, xws_hbm, dinv_hbm, zrow_hbm,
                       out_hbm, srcb, dstb, typb, gix_v, fix_v, sv_v, dsts,
                       rows_v, isem, gsem, ssem, csem, acc):
        c = lax.axis_index("c")
        s = lax.axis_index("s")
        wid = s * NC + c
        @pl.when(s < ncopy)
        def _():
            pltpu.sync_copy(zrow_hbm, acc.at[pl.ds(s * rps, rps)])
        plsc.subcore_barrier()

        ebase = wid * epw

        def fetch_idx(base, b):
            pltpu.async_copy(pk_hbm.at[pl.ds(base, CH)], srcb[b], isem[b])
            pltpu.async_copy(pk_hbm.at[pl.ds(E + base, CH)], dstb[b], isem[b])
            pltpu.async_copy(pk_hbm.at[pl.ds(2 * E + base, CH)], typb[b], isem[b])

        def wait_idx(b):
            pltpu.make_async_copy(pk_hbm.at[pl.ds(0, CH)], srcb[b], isem[b]).wait()
            pltpu.make_async_copy(pk_hbm.at[pl.ds(0, CH)], dstb[b], isem[b]).wait()
            pltpu.make_async_copy(pk_hbm.at[pl.ds(0, CH)], typb[b], isem[b]).wait()

        fetch_idx(ebase, 0)

        def halfstep(i, b):
            @pl.when(i <= nch - 1)
            def _():
                wait_idx(b)

                for k in range(CH // L):
                    sl = pl.ds(k * L, L)
                    t = typb[b][sl]
                    gix_v[b][sl] = t * N + srcb[b][sl]
                    fix_v[b][sl] = t * N + dstb[b][sl]
                    dsts[b][sl] = dstb[b][sl]
                pltpu.async_copy(xws_hbm.at[gix_v[b]], rows_v[b], gsem[b])
                pltpu.async_copy(dinv_hbm.at[fix_v[b]], sv_v[b], ssem[b])

            @pl.when(jnp.logical_and(i >= 2, i <= nch + 1))
            def _():
                o = (b + NB - 2) % NB
                ov = rows_v[o]
                sv = sv_v[o]
                pltpu.make_async_copy(xws_hbm.at[pl.ds(0, CH)],
                                      ov, gsem[o]).wait()
                pltpu.make_async_copy(pk_hbm.at[pl.ds(0, CH)],
                                      sv, ssem[o]).wait()
                for k in range(CH // L):
                    scales = sv[pl.ds(k * L, L)]
                    for e in range(L):
                        row = k * L + e
                        s_e = scales[e]
                        for j in range(OUT // L):
                            sl = pl.ds(j * L, L)
                            ov[row, sl] = ov[row, sl] * s_e
                pltpu.sync_copy(ov, acc.at[dsts[o]], add=True)

            @pl.when(i <= nch - 2)
            def _():
                fetch_idx(ebase + (i + 1) * CH, (b + 1) % NB)

        def quad(p, carry):
            for b in range(NB):
                halfstep(NB * p + b, b)
            return carry

        lax.fori_loop(0, niter // NB, quad, 0)
        plsc.subcore_barrier()

        @pl.when(s < ncopy)
        def _():
            pltpu.sync_copy(acc.at[pl.ds(s * rps, rps)],
                            out_hbm.at[c, pl.ds(s * rps, rps)])

    return scatter_kernel


def _final_body(accp_ref, self_ref, bias_ref, out_ref):
    out_ref[...] = (accp_ref[0] + accp_ref[1] + self_ref[0]
                    + bias_ref[...][None, :])


def _make_final(N, OUT, R, blkn):
    return pl.pallas_call(
        _final_body,
        grid=(N // blkn,),
        in_specs=[
            pl.BlockSpec((NC, blkn, OUT), lambda i: (0, i, 0)),
            pl.BlockSpec((1, blkn, OUT), lambda i: (R, i, 0)),
            pl.BlockSpec((OUT,), lambda i: (0,)),
        ],
        out_specs=pl.BlockSpec((blkn, OUT), lambda i: (i, 0)),
        out_shape=jax.ShapeDtypeStruct((N, OUT), jnp.float32),
    )


def kernel(x, edge_index, edge_type, coeff, bases, h_bias, loop_weight):
    N, IN = x.shape
    R, B = coeff.shape
    OUT = bases.shape[2]
    E = edge_type.shape[0]
    RN = R * N
    assert E % (NW * CH) == 0 and N % NS == 0 and OUT == 128

    packed = jnp.concatenate(
        [edge_index[0], edge_index[1], edge_type.astype(jnp.int32)])

    xws = _make_xw(N, IN, OUT, R, B, blkn=2000)(coeff, bases, loop_weight, x)
    degp = _make_deg(N, E, RN)(packed, jnp.zeros((RN,), jnp.float32))
    dinv = _make_dinv(RN)(degp)
    accp = _make_scatter(N, E, OUT, RN)(
        packed, xws.reshape((R + 1) * N, OUT), dinv,
        jnp.zeros((N // 10, OUT), jnp.float32))
    h = _make_final(N, OUT, R, blkn=2000)(accp, xws, h_bias)
    return h


# speedup vs baseline: 1.2854x; 1.2854x over previous
import functools

import jax
import jax.numpy as jnp
from jax import lax
from jax.experimental import pallas as pl
from jax.experimental.pallas import tpu as pltpu
from jax.experimental.pallas import tpu_sc as plsc

NC, NS, L = 2, 16, 16
NW = NC * NS
CH = 80
NB = 4


def _xw_body(coeff_ref, bases_ref, loop_ref, x_ref, out_ref):
    r = pl.program_id(0)
    R, B = coeff_ref.shape
    rc = jnp.minimum(r, R - 1)
    W = coeff_ref[rc, 0] * bases_ref[0]
    for b in range(1, B):
        W = W + coeff_ref[rc, b] * bases_ref[b]
    W = jnp.where(r < R, W, loop_ref[...])
    out_ref[0] = jnp.dot(x_ref[...], W, preferred_element_type=jnp.float32)


def _make_xw(N, IN, OUT, R, B, blkn):
    grid = (R + 1, N // blkn)
    return pl.pallas_call(
        _xw_body,
        grid=grid,
        in_specs=[
            pl.BlockSpec(memory_space=pltpu.SMEM),
            pl.BlockSpec((B, IN, OUT), lambda r, i: (0, 0, 0)),
            pl.BlockSpec((IN, OUT), lambda r, i: (0, 0)),
            pl.BlockSpec((blkn, IN), lambda r, i: (i, 0)),
        ],
        out_specs=pl.BlockSpec((1, blkn, OUT), lambda r, i: (r, i, 0)),
        out_shape=jax.ShapeDtypeStruct((R + 1, N, OUT), jnp.float32),
    )


def _make_deg(N, E, RN):
    epw = E // NW
    mesh = plsc.VectorSubcoreMesh(core_axis_name="c", subcore_axis_name="s",
                                  num_cores=NC, num_subcores=NS)

    @functools.partial(
        pl.kernel,
        out_type=jax.ShapeDtypeStruct((NW, RN), jnp.float32),
        mesh=mesh,
        compiler_params=pltpu.CompilerParams(needs_layout_passes=False),
        scratch_types=[
            pltpu.VMEM((epw,), jnp.int32),
            pltpu.VMEM((epw,), jnp.int32),
            pltpu.VMEM((RN,), jnp.float32),
        ],
    )
    def deg_kernel(pk_hbm, zeros_hbm, out_hbm, typ_v, dst_v, deg_v):
        c = lax.axis_index("c")
        s = lax.axis_index("s")
        wid = s * NC + c
        ebase = wid * epw
        pltpu.sync_copy(pk_hbm.at[pl.ds(2 * E + ebase, epw)], typ_v)
        pltpu.sync_copy(pk_hbm.at[pl.ds(E + ebase, epw)], dst_v)
        pltpu.sync_copy(zeros_hbm, deg_v)
        ones = jnp.ones((L,), jnp.float32)
        gpi = 5

        def step(i, carry):
            for g in range(gpi):
                sl = pl.ds(i * (gpi * L) + g * L, L)
                idx = typ_v[sl] * N + dst_v[sl]
                plsc.addupdate_scatter(deg_v, [idx], ones)
            return carry

        lax.fori_loop(0, epw // (gpi * L), step, 0)
        pltpu.sync_copy(deg_v, out_hbm.at[wid])

    return deg_kernel


def _dinv_body(degp_ref, out_ref):
    total = jnp.sum(degp_ref[...], axis=0)
    out_ref[...] = 1.0 / jnp.maximum(total, 1.0)


def _make_dinv(RN):
    return pl.pallas_call(
        _dinv_body,
        out_shape=jax.ShapeDtypeStruct((RN,), jnp.float32),
    )


def _make_scatter(N, E, OUT, RN):
    epw = E // NW
    nch = epw // CH
    ncopy = 10
    rps = N // ncopy
    mesh = plsc.VectorSubcoreMesh(core_axis_name="c", subcore_axis_name="s",
                                  num_cores=NC, num_subcores=NS)

    @functools.partial(
        pl.kernel,
        out_type=jax.ShapeDtypeStruct((NC, N, OUT), jnp.float32),
        mesh=mesh,
        compiler_params=pltpu.CompilerParams(needs_layout_passes=False),
        scratch_types=[
            [pltpu.VMEM((3 * CH,), jnp.int32)] * 2,
            [pltpu.VMEM((CH,), jnp.int32)] * 2,
            [pltpu.VMEM((CH,), jnp.int32)] * 2,
            [pltpu.VMEM((CH,), jnp.int32)] * 2,
            [pltpu.VMEM((CH,), jnp.float32)] * 2,
            [pltpu.VMEM((CH, OUT), jnp.float32)] * 2,
            [pltpu.SemaphoreType.DMA] * 2,
            [pltpu.SemaphoreType.DMA] * 2,
            [pltpu.SemaphoreType.DMA] * 2,
            [pltpu.SemaphoreType.DMA] * 2,
            pltpu.VMEM_SHARED((N, OUT), jnp.float32),
        ],
    )
    def scatter_kernel(pk3_hbm, xws_hbm, dinv_hbm, zrow_hbm,
                       out_hbm, ib, gix_v, fix_v, dsts, sv_v, rows_v,
                       isem, gsem, ssem, csem, acc):
        c = lax.axis_index("c")
        s = lax.axis_index("s")
        wid = s * NC + c
        @pl.when(s < ncopy)
        def _():
            pltpu.sync_copy(zrow_hbm, acc.at[pl.ds(s * rps, rps)])
        plsc.subcore_barrier()

        cbase = wid * nch

        def fetch_idx(ci, b):
            pltpu.async_copy(pk3_hbm.at[pl.ds(ci * (3 * CH), 3 * CH)],
                             ib[b], isem[b])

        fetch_idx(cbase, 0)

        def halfstep(i, b):
            @pl.when(i <= nch - 1)
            def _():
                pltpu.make_async_copy(pk3_hbm.at[pl.ds(0, 3 * CH)],
                                      ib[b], isem[b]).wait()

                @pl.when(i >= 2)
                def _():
                    pltpu.make_async_copy(xws_hbm.at[pl.ds(0, CH)],
                                          rows_v[b], csem[b]).wait()
                for k in range(CH // L):
                    sl = pl.ds(k * L, L)
                    t = ib[b][pl.ds(2 * CH + k * L, L)]
                    gix_v[b][sl] = t * N + ib[b][pl.ds(k * L, L)]
                    fix_v[b][sl] = t * N + ib[b][pl.ds(CH + k * L, L)]
                    dsts[b][sl] = ib[b][pl.ds(CH + k * L, L)]
                pltpu.async_copy(xws_hbm.at[gix_v[b]], rows_v[b], gsem[b])
                pltpu.async_copy(dinv_hbm.at[fix_v[b]], sv_v[b], ssem[b])

            @pl.when(jnp.logical_and(i >= 1, i <= nch))
            def _():
                o = 1 - b
                ov = rows_v[o]
                sv = sv_v[o]
                pltpu.make_async_copy(xws_hbm.at[pl.ds(0, CH)],
                                      ov, gsem[o]).wait()
                pltpu.make_async_copy(dinv_hbm.at[pl.ds(0, CH)],
                                      sv, ssem[o]).wait()
                for k in range(CH // L):
                    scales = sv[pl.ds(k * L, L)]
                    for e in range(L):
                        row = k * L + e
                        s_e = scales[e]
                        for j in range(OUT // L):
                            sl = pl.ds(j * L, L)
                            ov[row, sl] = ov[row, sl] * s_e
                pltpu.async_copy(ov, acc.at[dsts[o]], csem[o], add=True)

            @pl.when(i <= nch - 2)
            def _():
                fetch_idx(cbase + i + 1, 1 - b)

        def pair(p, carry):
            halfstep(2 * p, 0)
            halfstep(2 * p + 1, 1)
            return carry

        lax.fori_loop(0, (nch + 2) // 2, pair, 0)
        for d in range(1, 3):
            m = (nch - d) % 2
            pltpu.make_async_copy(xws_hbm.at[pl.ds(0, CH)],
                                  rows_v[m], csem[m]).wait()
        plsc.subcore_barrier()

        @pl.when(s < ncopy)
        def _():
            pltpu.sync_copy(acc.at[pl.ds(s * rps, rps)],
                            out_hbm.at[c, pl.ds(s * rps, rps)])

    return scatter_kernel


def _final_body(accp_ref, self_ref, bias_ref, out_ref):
    out_ref[...] = (accp_ref[0] + accp_ref[1] + self_ref[0]
                    + bias_ref[...][None, :])


def _make_final(N, OUT, R, blkn):
    return pl.pallas_call(
        _final_body,
        grid=(N // blkn,),
        in_specs=[
            pl.BlockSpec((NC, blkn, OUT), lambda i: (0, i, 0)),
            pl.BlockSpec((1, blkn, OUT), lambda i: (R, i, 0)),
            pl.BlockSpec((OUT,), lambda i: (0,)),
        ],
        out_specs=pl.BlockSpec((blkn, OUT), lambda i: (i, 0)),
        out_shape=jax.ShapeDtypeStruct((N, OUT), jnp.float32),
    )


def kernel(x, edge_index, edge_type, coeff, bases, h_bias, loop_weight):
    N, IN = x.shape
    R, B = coeff.shape
    OUT = bases.shape[2]
    E = edge_type.shape[0]
    RN = R * N
    assert E % (NW * CH) == 0 and N % NS == 0 and OUT == 128

    packed = jnp.concatenate(
        [edge_index[0], edge_index[1], edge_type.astype(jnp.int32)])

    xws = _make_xw(N, IN, OUT, R, B, blkn=2000)(coeff, bases, loop_weight, x)
    degp = _make_deg(N, E, RN)(packed, jnp.zeros((RN,), jnp.float32))
    dinv = _make_dinv(RN)(degp)
    packed3 = packed.reshape(3, E // CH, CH).transpose(1, 0, 2).reshape(-1)
    accp = _make_scatter(N, E, OUT, RN)(
        packed3, xws.reshape((R + 1) * N, OUT), dinv,
        jnp.zeros((N // 10, OUT), jnp.float32))
    h = _make_final(N, OUT, R, blkn=2000)(accp, xws, h_bias)
    return h


# speedup vs baseline: 1.4407x; 1.1209x over previous
import functools

import jax
import jax.numpy as jnp
from jax import lax
from jax.experimental import pallas as pl
from jax.experimental.pallas import tpu as pltpu
from jax.experimental.pallas import tpu_sc as plsc

NC, NS, L = 2, 16, 16
NW = NC * NS
CH = 80
NB = 4


def _xw_body(coeff_ref, bases_ref, loop_ref, x_ref, out_ref):
    r = pl.program_id(0)
    R, B = coeff_ref.shape
    rc = jnp.minimum(r, R - 1)
    W = coeff_ref[rc, 0] * bases_ref[0]
    for b in range(1, B):
        W = W + coeff_ref[rc, b] * bases_ref[b]
    W = jnp.where(r < R, W, loop_ref[...])
    out_ref[0] = jnp.dot(x_ref[...], W, preferred_element_type=jnp.float32)


def _make_xw(N, IN, OUT, R, B, blkn):
    grid = (R + 1, N // blkn)
    return pl.pallas_call(
        _xw_body,
        grid=grid,
        in_specs=[
            pl.BlockSpec(memory_space=pltpu.SMEM),
            pl.BlockSpec((B, IN, OUT), lambda r, i: (0, 0, 0)),
            pl.BlockSpec((IN, OUT), lambda r, i: (0, 0)),
            pl.BlockSpec((blkn, IN), lambda r, i: (i, 0)),
        ],
        out_specs=pl.BlockSpec((1, blkn, OUT), lambda r, i: (r, i, 0)),
        out_shape=jax.ShapeDtypeStruct((R + 1, N, OUT), jnp.float32),
    )


def _make_deg(N, E, RN):
    epw = E // NW
    mesh = plsc.VectorSubcoreMesh(core_axis_name="c", subcore_axis_name="s",
                                  num_cores=NC, num_subcores=NS)

    @functools.partial(
        pl.kernel,
        out_type=jax.ShapeDtypeStruct((NW, RN), jnp.float32),
        mesh=mesh,
        compiler_params=pltpu.CompilerParams(needs_layout_passes=False),
        scratch_types=[
            pltpu.VMEM((epw,), jnp.int32),
            pltpu.VMEM((epw,), jnp.int32),
            pltpu.VMEM((RN,), jnp.float32),
        ],
    )
    def deg_kernel(pk_hbm, zeros_hbm, out_hbm, typ_v, dst_v, deg_v):
        c = lax.axis_index("c")
        s = lax.axis_index("s")
        wid = s * NC + c
        ebase = wid * epw
        pltpu.sync_copy(pk_hbm.at[pl.ds(2 * E + ebase, epw)], typ_v)
        pltpu.sync_copy(pk_hbm.at[pl.ds(E + ebase, epw)], dst_v)
        pltpu.sync_copy(zeros_hbm, deg_v)
        ones = jnp.ones((L,), jnp.float32)
        gpi = 5

        def step(i, carry):
            for g in range(gpi):
                sl = pl.ds(i * (gpi * L) + g * L, L)
                idx = typ_v[sl] * N + dst_v[sl]
                plsc.addupdate_scatter(deg_v, [idx], ones)
            return carry

        lax.fori_loop(0, epw // (gpi * L), step, 0)
        pltpu.sync_copy(deg_v, out_hbm.at[wid])

    return deg_kernel


def _dinv_body(degp_ref, out_ref):
    total = jnp.sum(degp_ref[...], axis=0)
    out_ref[...] = 1.0 / jnp.maximum(total, 1.0)


def _make_dinv(RN):
    return pl.pallas_call(
        _dinv_body,
        out_shape=jax.ShapeDtypeStruct((RN,), jnp.float32),
    )


def _make_scatter(N, E, OUT, RN):
    epw = E // NW
    nch = epw // CH
    ncopy = 10
    rps = N // ncopy
    mesh = plsc.VectorSubcoreMesh(core_axis_name="c", subcore_axis_name="s",
                                  num_cores=NC, num_subcores=NS)

    @functools.partial(
        pl.kernel,
        out_type=jax.ShapeDtypeStruct((NC, N, OUT), jnp.float32),
        mesh=mesh,
        compiler_params=pltpu.CompilerParams(needs_layout_passes=False),
        scratch_types=[
            [pltpu.VMEM((3 * CH,), jnp.int32)] * 2,
            [pltpu.VMEM((CH,), jnp.int32)] * 2,
            [pltpu.VMEM((CH,), jnp.int32)] * 2,
            [pltpu.VMEM((CH,), jnp.int32)] * 2,
            [pltpu.VMEM((CH,), jnp.float32)] * 2,
            [pltpu.VMEM((CH, OUT), jnp.float32)] * 2,
            [pltpu.SemaphoreType.DMA] * 2,
            [pltpu.SemaphoreType.DMA] * 2,
            [pltpu.SemaphoreType.DMA] * 2,
            [pltpu.SemaphoreType.DMA] * 2,
            pltpu.VMEM_SHARED((N, OUT), jnp.float32),
        ],
    )
    def scatter_kernel(pk3_hbm, xws_hbm, dinv_hbm, zrow_hbm,
                       out_hbm, ib, gix_v, fix_v, dsts, sv_v, rows_v,
                       isem, gsem, ssem, csem, acc):
        c = lax.axis_index("c")
        s = lax.axis_index("s")
        wid = s * NC + c
        @pl.when(s < ncopy)
        def _():
            pltpu.sync_copy(zrow_hbm, acc.at[pl.ds(s * rps, rps)])
        plsc.subcore_barrier()

        cbase = wid * nch

        def fetch_idx(ci, b):
            pltpu.async_copy(pk3_hbm.at[pl.ds(ci * (3 * CH), 3 * CH)],
                             ib[b], isem[b])

        fetch_idx(cbase, 0)

        def halfstep(i, b):
            @pl.when(i <= nch - 1)
            def _():
                pltpu.make_async_copy(pk3_hbm.at[pl.ds(0, 3 * CH)],
                                      ib[b], isem[b]).wait()

                @pl.when(i >= 2)
                def _():
                    pltpu.make_async_copy(xws_hbm.at[pl.ds(0, CH)],
                                          rows_v[b], csem[b]).wait()
                for k in range(CH // L):
                    sl = pl.ds(k * L, L)
                    t = ib[b][pl.ds(2 * CH + k * L, L)]
                    gix_v[b][sl] = t * N + ib[b][pl.ds(k * L, L)]
                    fix_v[b][sl] = t * N + ib[b][pl.ds(CH + k * L, L)]
                    dsts[b][sl] = ib[b][pl.ds(CH + k * L, L)]
                pltpu.async_copy(xws_hbm.at[gix_v[b]], rows_v[b], gsem[b])
                pltpu.async_copy(dinv_hbm.at[fix_v[b]], sv_v[b], ssem[b])

            @pl.when(jnp.logical_and(i >= 1, i <= nch))
            def _():
                o = 1 - b
                ov = rows_v[o]
                sv = sv_v[o]
                pltpu.make_async_copy(xws_hbm.at[pl.ds(0, CH)],
                                      ov, gsem[o]).wait()
                pltpu.make_async_copy(dinv_hbm.at[pl.ds(0, CH)],
                                      sv, ssem[o]).wait()
                for k in range(CH // L):
                    scales = sv[pl.ds(k * L, L)]
                    for e in range(L):
                        row = k * L + e
                        s_e = scales[e]
                        for j in range(OUT // L):
                            sl = pl.ds(j * L, L)
                            ov[row, sl] = ov[row, sl] * s_e
                pltpu.async_copy(ov, acc.at[dsts[o]], csem[o], add=True)

            @pl.when(i <= nch - 2)
            def _():
                fetch_idx(cbase + i + 1, 1 - b)

        def pair(p, carry):
            halfstep(2 * p, 0)
            halfstep(2 * p + 1, 1)
            return carry

        lax.fori_loop(0, (nch + 2) // 2, pair, 0)
        for d in range(1, 3):
            m = (nch - d) % 2
            pltpu.make_async_copy(xws_hbm.at[pl.ds(0, CH)],
                                  rows_v[m], csem[m]).wait()
        plsc.subcore_barrier()

        @pl.when(s < ncopy)
        def _():
            pltpu.sync_copy(acc.at[pl.ds(s * rps, rps)],
                            out_hbm.at[c, pl.ds(s * rps, rps)])

    return scatter_kernel


def _final_body(accp_ref, self_ref, bias_ref, out_ref):
    out_ref[...] = (accp_ref[0] + accp_ref[1] + self_ref[0]
                    + bias_ref[...][None, :])


def _make_final(N, OUT, R, blkn):
    return pl.pallas_call(
        _final_body,
        grid=(N // blkn,),
        in_specs=[
            pl.BlockSpec((NC, blkn, OUT), lambda i: (0, i, 0)),
            pl.BlockSpec((1, blkn, OUT), lambda i: (R, i, 0)),
            pl.BlockSpec((OUT,), lambda i: (0,)),
        ],
        out_specs=pl.BlockSpec((blkn, OUT), lambda i: (i, 0)),
        out_shape=jax.ShapeDtypeStruct((N, OUT), jnp.float32),
    )


def kernel(x, edge_index, edge_type, coeff, bases, h_bias, loop_weight):
    N, IN = x.shape
    R, B = coeff.shape
    OUT = bases.shape[2]
    E = edge_type.shape[0]
    RN = R * N
    assert E % (NW * CH) == 0 and N % NS == 0 and OUT == 128

    packed = jnp.concatenate(
        [edge_index[0], edge_index[1], edge_type.astype(jnp.int32)])

    xws = _make_xw(N, IN, OUT, R, B, blkn=N)(coeff, bases, loop_weight, x)
    degp = _make_deg(N, E, RN)(packed, jnp.zeros((RN,), jnp.float32))
    dinv = _make_dinv(RN)(degp)
    packed3 = packed.reshape(3, E // CH, CH).transpose(1, 0, 2).reshape(-1)
    accp = _make_scatter(N, E, OUT, RN)(
        packed3, xws.reshape((R + 1) * N, OUT), dinv,
        jnp.zeros((N // 10, OUT), jnp.float32))
    h = _make_final(N, OUT, R, blkn=N)(accp, xws, h_bias)
    return h


# speedup vs baseline: 1.6782x; 1.1648x over previous
import functools

import jax
import jax.numpy as jnp
from jax import lax
from jax.experimental import pallas as pl
from jax.experimental.pallas import tpu as pltpu
from jax.experimental.pallas import tpu_sc as plsc

NC, NS, L = 2, 16, 16
NW = NC * NS
CH = 80
NB = 4


def _xw_body(coeff_ref, bases_ref, loop_ref, x_ref, out_ref):
    r = pl.program_id(0)
    R, B = coeff_ref.shape
    rc = jnp.minimum(r, R - 1)
    W = coeff_ref[rc, 0] * bases_ref[0]
    for b in range(1, B):
        W = W + coeff_ref[rc, b] * bases_ref[b]
    W = jnp.where(r < R, W, loop_ref[...])
    out_ref[...] = jnp.dot(x_ref[...], W, preferred_element_type=jnp.float32)


def _make_xw(N, IN, OUT, R, B, blkn):
    grid = (R + 1, N // blkn)
    return pl.pallas_call(
        _xw_body,
        grid=grid,
        in_specs=[
            pl.BlockSpec(memory_space=pltpu.SMEM),
            pl.BlockSpec((B, IN, OUT), lambda r, i: (0, 0, 0)),
            pl.BlockSpec((IN, OUT), lambda r, i: (0, 0)),
            pl.BlockSpec((blkn, IN), lambda r, i: (i, 0)),
        ],
        out_specs=pl.BlockSpec((blkn, OUT), lambda r, i: (r, 0)),
        out_shape=jax.ShapeDtypeStruct(((R + 1) * N, OUT), jnp.float32),
    )


def _make_deg(N, E, RN):
    epw = E // NW
    mesh = plsc.VectorSubcoreMesh(core_axis_name="c", subcore_axis_name="s",
                                  num_cores=NC, num_subcores=NS)

    @functools.partial(
        pl.kernel,
        out_type=jax.ShapeDtypeStruct((NW, RN), jnp.float32),
        mesh=mesh,
        compiler_params=pltpu.CompilerParams(needs_layout_passes=False),
        scratch_types=[
            pltpu.VMEM((epw,), jnp.int32),
            pltpu.VMEM((epw,), jnp.int32),
            pltpu.VMEM((RN,), jnp.float32),
        ],
    )
    def deg_kernel(ei_hbm, tp_hbm, zeros_hbm, out_hbm, typ_v, dst_v, deg_v):
        c = lax.axis_index("c")
        s = lax.axis_index("s")
        wid = s * NC + c
        ebase = wid * epw
        pltpu.sync_copy(tp_hbm.at[pl.ds(ebase, epw)], typ_v)
        pltpu.sync_copy(ei_hbm.at[pl.ds(E + ebase, epw)], dst_v)
        pltpu.sync_copy(zeros_hbm, deg_v)
        ones = jnp.ones((L,), jnp.float32)
        gpi = 5

        def step(i, carry):
            for g in range(gpi):
                sl = pl.ds(i * (gpi * L) + g * L, L)
                idx = typ_v[sl] * N + dst_v[sl]
                plsc.addupdate_scatter(deg_v, [idx], ones)
            return carry

        lax.fori_loop(0, epw // (gpi * L), step, 0)
        pltpu.sync_copy(deg_v, out_hbm.at[wid])

    return deg_kernel


def _dinv_body(degp_ref, out_ref):
    total = jnp.sum(degp_ref[...], axis=0)
    out_ref[...] = 1.0 / jnp.maximum(total, 1.0)


def _make_dinv(RN):
    return pl.pallas_call(
        _dinv_body,
        out_shape=jax.ShapeDtypeStruct((RN,), jnp.float32),
    )


def _make_scatter(N, E, OUT, RN):
    epw = E // NW
    nch = epw // CH
    ncopy = 10
    rps = N // ncopy
    mesh = plsc.VectorSubcoreMesh(core_axis_name="c", subcore_axis_name="s",
                                  num_cores=NC, num_subcores=NS)

    @functools.partial(
        pl.kernel,
        out_type=jax.ShapeDtypeStruct((NC, N, OUT), jnp.float32),
        mesh=mesh,
        compiler_params=pltpu.CompilerParams(needs_layout_passes=False),
        scratch_types=[
            [pltpu.VMEM((CH,), jnp.int32)] * 2,
            [pltpu.VMEM((CH,), jnp.int32)] * 2,
            [pltpu.VMEM((CH,), jnp.int32)] * 2,
            [pltpu.VMEM((CH,), jnp.int32)] * 2,
            [pltpu.VMEM((CH,), jnp.int32)] * 2,
            [pltpu.VMEM((CH,), jnp.int32)] * 2,
            [pltpu.VMEM((CH,), jnp.float32)] * 2,
            [pltpu.VMEM((CH, OUT), jnp.float32)] * 2,
            [pltpu.SemaphoreType.DMA] * 2,
            [pltpu.SemaphoreType.DMA] * 2,
            [pltpu.SemaphoreType.DMA] * 2,
            [pltpu.SemaphoreType.DMA] * 2,
            pltpu.VMEM_SHARED((N, OUT), jnp.float32),
        ],
    )
    def scatter_kernel(ei_hbm, tp_hbm, xws_hbm, dinv_hbm, zrow_hbm,
                       out_hbm, srcb, dstb, typb, gix_v, fix_v, dsts, sv_v,
                       rows_v, isem, gsem, ssem, csem, acc):
        c = lax.axis_index("c")
        s = lax.axis_index("s")
        wid = s * NC + c
        @pl.when(s < ncopy)
        def _():
            pltpu.sync_copy(zrow_hbm, acc.at[pl.ds(s * rps, rps)])
        plsc.subcore_barrier()

        ebase = wid * epw

        def fetch_idx(base, b):
            pltpu.async_copy(ei_hbm.at[pl.ds(base, CH)], srcb[b], isem[b])
            pltpu.async_copy(ei_hbm.at[pl.ds(E + base, CH)], dstb[b], isem[b])
            pltpu.async_copy(tp_hbm.at[pl.ds(base, CH)], typb[b], isem[b])

        fetch_idx(ebase, 0)

        def halfstep(i, b):
            @pl.when(i <= nch - 1)
            def _():
                pltpu.make_async_copy(ei_hbm.at[pl.ds(0, CH)],
                                      srcb[b], isem[b]).wait()
                pltpu.make_async_copy(ei_hbm.at[pl.ds(0, CH)],
                                      dstb[b], isem[b]).wait()
                pltpu.make_async_copy(tp_hbm.at[pl.ds(0, CH)],
                                      typb[b], isem[b]).wait()

                @pl.when(i >= 2)
                def _():
                    pltpu.make_async_copy(xws_hbm.at[pl.ds(0, CH)],
                                          rows_v[b], csem[b]).wait()
                for k in range(CH // L):
                    sl = pl.ds(k * L, L)
                    t = typb[b][sl]
                    gix_v[b][sl] = t * N + srcb[b][sl]
                    fix_v[b][sl] = t * N + dstb[b][sl]
                    dsts[b][sl] = dstb[b][sl]
                pltpu.async_copy(xws_hbm.at[gix_v[b]], rows_v[b], gsem[b])
                pltpu.async_copy(dinv_hbm.at[fix_v[b]], sv_v[b], ssem[b])

            @pl.when(jnp.logical_and(i >= 1, i <= nch))
            def _():
                o = 1 - b
                ov = rows_v[o]
                sv = sv_v[o]
                pltpu.make_async_copy(xws_hbm.at[pl.ds(0, CH)],
                                      ov, gsem[o]).wait()
                pltpu.make_async_copy(dinv_hbm.at[pl.ds(0, CH)],
                                      sv, ssem[o]).wait()
                for k in range(CH // L):
                    scales = sv[pl.ds(k * L, L)]
                    for e in range(L):
                        row = k * L + e
                        s_e = scales[e]
                        for j in range(OUT // L):
                            sl = pl.ds(j * L, L)
                            ov[row, sl] = ov[row, sl] * s_e
                pltpu.async_copy(ov, acc.at[dsts[o]], csem[o], add=True)

            @pl.when(i <= nch - 2)
            def _():
                fetch_idx(ebase + (i + 1) * CH, 1 - b)

        def pair(p, carry):
            halfstep(2 * p, 0)
            halfstep(2 * p + 1, 1)
            return carry

        lax.fori_loop(0, (nch + 2) // 2, pair, 0)
        for d in range(1, 3):
            m = (nch - d) % 2
            pltpu.make_async_copy(xws_hbm.at[pl.ds(0, CH)],
                                  rows_v[m], csem[m]).wait()
        plsc.subcore_barrier()

        @pl.when(s < ncopy)
        def _():
            pltpu.sync_copy(acc.at[pl.ds(s * rps, rps)],
                            out_hbm.at[c, pl.ds(s * rps, rps)])

    return scatter_kernel


def _final_body(accp_ref, self_ref, bias_ref, out_ref):
    out_ref[...] = (accp_ref[0] + accp_ref[1] + self_ref[...]
                    + bias_ref[...][None, :])


def _make_final(N, OUT, R, blkn):
    return pl.pallas_call(
        _final_body,
        grid=(N // blkn,),
        in_specs=[
            pl.BlockSpec((NC, blkn, OUT), lambda i: (0, i, 0)),
            pl.BlockSpec((blkn, OUT), lambda i: (R, 0)),
            pl.BlockSpec((OUT,), lambda i: (0,)),
        ],
        out_specs=pl.BlockSpec((blkn, OUT), lambda i: (i, 0)),
        out_shape=jax.ShapeDtypeStruct((N, OUT), jnp.float32),
    )


def kernel(x, edge_index, edge_type, coeff, bases, h_bias, loop_weight):
    N, IN = x.shape
    R, B = coeff.shape
    OUT = bases.shape[2]
    E = edge_type.shape[0]
    RN = R * N
    assert E % (NW * CH) == 0 and N % NS == 0 and OUT == 128

    ei_flat = edge_index.reshape(2 * E)
    typ = edge_type.astype(jnp.int32)

    xws = _make_xw(N, IN, OUT, R, B, blkn=N)(coeff, bases, loop_weight, x)
    degp = _make_deg(N, E, RN)(ei_flat, typ, jnp.zeros((RN,), jnp.float32))
    dinv = _make_dinv(RN)(degp)
    accp = _make_scatter(N, E, OUT, RN)(
        ei_flat, typ, xws, dinv, jnp.zeros((N // 10, OUT), jnp.float32))
    h = _make_final(N, OUT, R, blkn=N)(accp, xws, h_bias)
    return h


# speedup vs baseline: 1.6800x; 1.0011x over previous
import functools

import jax
import jax.numpy as jnp
from jax import lax
from jax.experimental import pallas as pl
from jax.experimental.pallas import tpu as pltpu
from jax.experimental.pallas import tpu_sc as plsc

NC, NS, L = 2, 16, 16
NW = NC * NS
CH = 80


def _xw_body(coeff_ref, bases_ref, loop_ref, x_ref, out_ref):
    r = pl.program_id(0)
    R, B = coeff_ref.shape
    rc = jnp.minimum(r, R - 1)
    W = coeff_ref[rc, 0] * bases_ref[0]
    for b in range(1, B):
        W = W + coeff_ref[rc, b] * bases_ref[b]
    W = jnp.where(r < R, W, loop_ref[...])
    out_ref[...] = jnp.dot(x_ref[...], W, preferred_element_type=jnp.float32)


def _make_xw(N, IN, OUT, R, B, blkn):
    grid = (R + 1, N // blkn)
    return pl.pallas_call(
        _xw_body,
        grid=grid,
        in_specs=[
            pl.BlockSpec(memory_space=pltpu.SMEM),
            pl.BlockSpec((B, IN, OUT), lambda r, i: (0, 0, 0)),
            pl.BlockSpec((IN, OUT), lambda r, i: (0, 0)),
            pl.BlockSpec((blkn, IN), lambda r, i: (i, 0)),
        ],
        out_specs=pl.BlockSpec((blkn, OUT), lambda r, i: (r, 0)),
        out_shape=jax.ShapeDtypeStruct(((R + 1) * N, OUT), jnp.float32),
    )


def _make_deg(N, E, RN):
    epw = E // NW
    mesh = plsc.VectorSubcoreMesh(core_axis_name="c", subcore_axis_name="s",
                                  num_cores=NC, num_subcores=NS)

    @functools.partial(
        pl.kernel,
        out_type=jax.ShapeDtypeStruct((NW, RN), jnp.float32),
        mesh=mesh,
        compiler_params=pltpu.CompilerParams(needs_layout_passes=False),
        scratch_types=[
            pltpu.VMEM((epw,), jnp.int32),
            pltpu.VMEM((epw,), jnp.int32),
            pltpu.VMEM((RN,), jnp.float32),
        ],
    )
    def deg_kernel(ei_hbm, tp_hbm, zeros_hbm, out_hbm, typ_v, dst_v, deg_v):
        c = lax.axis_index("c")
        s = lax.axis_index("s")
        wid = s * NC + c
        ebase = wid * epw
        pltpu.sync_copy(tp_hbm.at[pl.ds(ebase, epw)], typ_v)
        pltpu.sync_copy(ei_hbm.at[pl.ds(E + ebase, epw)], dst_v)
        pltpu.sync_copy(zeros_hbm, deg_v)
        ones = jnp.ones((L,), jnp.float32)
        gpi = 5

        def step(i, carry):
            for g in range(gpi):
                sl = pl.ds(i * (gpi * L) + g * L, L)
                idx = typ_v[sl] * N + dst_v[sl]
                plsc.addupdate_scatter(deg_v, [idx], ones)
            return carry

        lax.fori_loop(0, epw // (gpi * L), step, 0)
        pltpu.sync_copy(deg_v, out_hbm.at[wid])

    return deg_kernel


def _dinv_body(degp_ref, out_ref):
    total = jnp.sum(degp_ref[...], axis=0)
    out_ref[...] = 1.0 / jnp.maximum(total, 1.0)


def _make_dinv(RN):
    return pl.pallas_call(
        _dinv_body,
        out_shape=jax.ShapeDtypeStruct((RN,), jnp.float32),
    )


def _make_scatter(N, E, OUT, RN):
    epw = E // NW
    nch = epw // CH
    ncopy = 10
    rps = N // ncopy
    mesh = plsc.VectorSubcoreMesh(core_axis_name="c", subcore_axis_name="s",
                                  num_cores=NC, num_subcores=NS)

    @functools.partial(
        pl.kernel,
        out_type=jax.ShapeDtypeStruct((NC, N, OUT), jnp.float32),
        mesh=mesh,
        compiler_params=pltpu.CompilerParams(needs_layout_passes=False),
        scratch_types=[
            [pltpu.VMEM((CH,), jnp.int32)] * 2,
            [pltpu.VMEM((CH,), jnp.int32)] * 2,
            [pltpu.VMEM((CH,), jnp.int32)] * 2,
            [pltpu.VMEM((CH,), jnp.int32)] * 2,
            [pltpu.VMEM((CH,), jnp.int32)] * 2,
            [pltpu.VMEM((CH,), jnp.int32)] * 2,
            [pltpu.VMEM((CH,), jnp.float32)] * 2,
            [pltpu.VMEM((CH, OUT), jnp.float32)] * 2,
            [pltpu.SemaphoreType.DMA] * 2,
            [pltpu.SemaphoreType.DMA] * 2,
            [pltpu.SemaphoreType.DMA] * 2,
            [pltpu.SemaphoreType.DMA] * 2,
            pltpu.VMEM_SHARED((N, OUT), jnp.float32),
        ],
    )
    def scatter_kernel(ei_hbm, tp_hbm, xws_hbm, dinv_hbm, zrow_hbm,
                       out_hbm, srcb, dstb, typb, gix_v, fix_v, dsts, sv_v,
                       rows_v, isem, gsem, ssem, csem, acc):
        c = lax.axis_index("c")
        s = lax.axis_index("s")
        wid = s * NC + c
        @pl.when(s < ncopy)
        def _():
            pltpu.sync_copy(zrow_hbm, acc.at[pl.ds(s * rps, rps)])
        plsc.subcore_barrier()

        ebase = wid * epw

        def fetch_idx(base, b):
            pltpu.async_copy(ei_hbm.at[pl.ds(base, CH)], srcb[b], isem[b])
            pltpu.async_copy(ei_hbm.at[pl.ds(E + base, CH)], dstb[b], isem[b])
            pltpu.async_copy(tp_hbm.at[pl.ds(base, CH)], typb[b], isem[b])

        fetch_idx(ebase, 0)

        def halfstep(i, b):
            @pl.when(i <= nch - 1)
            def _():
                pltpu.make_async_copy(ei_hbm.at[pl.ds(0, CH)],
                                      srcb[b], isem[b]).wait()
                pltpu.make_async_copy(ei_hbm.at[pl.ds(0, CH)],
                                      dstb[b], isem[b]).wait()
                pltpu.make_async_copy(tp_hbm.at[pl.ds(0, CH)],
                                      typb[b], isem[b]).wait()

                @pl.when(i >= 2)
                def _():
                    pltpu.make_async_copy(xws_hbm.at[pl.ds(0, CH)],
                                          rows_v[b], csem[b]).wait()
                for k in range(CH // L):
                    sl = pl.ds(k * L, L)
                    t = typb[b][sl]
                    gix_v[b][sl] = t * N + srcb[b][sl]
                    fix_v[b][sl] = t * N + dstb[b][sl]
                    dsts[b][sl] = dstb[b][sl]
                pltpu.async_copy(xws_hbm.at[gix_v[b]], rows_v[b], gsem[b])
                pltpu.async_copy(dinv_hbm.at[fix_v[b]], sv_v[b], ssem[b])

            @pl.when(jnp.logical_and(i >= 1, i <= nch))
            def _():
                o = 1 - b
                ov = rows_v[o]
                sv = sv_v[o]
                pltpu.make_async_copy(xws_hbm.at[pl.ds(0, CH)],
                                      ov, gsem[o]).wait()
                pltpu.make_async_copy(dinv_hbm.at[pl.ds(0, CH)],
                                      sv, ssem[o]).wait()
                for k in range(CH // L):
                    scales = sv[pl.ds(k * L, L)]
                    for e in range(L):
                        row = k * L + e
                        s_e = scales[e]
                        for j in range(OUT // L):
                            sl = pl.ds(j * L, L)
                            ov[row, sl] = ov[row, sl] * s_e
                pltpu.async_copy(ov, acc.at[dsts[o]], csem[o], add=True)

            @pl.when(i <= nch - 2)
            def _():
                fetch_idx(ebase + (i + 1) * CH, 1 - b)

        def pair(p, carry):
            halfstep(2 * p, 0)
            halfstep(2 * p + 1, 1)
            return carry

        lax.fori_loop(0, (nch + 2) // 2, pair, 0)
        for d in range(1, 3):
            m = (nch - d) % 2
            pltpu.make_async_copy(xws_hbm.at[pl.ds(0, CH)],
                                  rows_v[m], csem[m]).wait()
        plsc.subcore_barrier()

        @pl.when(s < ncopy)
        def _():
            pltpu.sync_copy(acc.at[pl.ds(s * rps, rps)],
                            out_hbm.at[c, pl.ds(s * rps, rps)])

    return scatter_kernel


def _final_body(accp_ref, self_ref, bias_ref, out_ref):
    out_ref[...] = (accp_ref[0] + accp_ref[1] + self_ref[...]
                    + bias_ref[...][None, :])


def _make_final(N, OUT, R, blkn):
    return pl.pallas_call(
        _final_body,
        grid=(N // blkn,),
        in_specs=[
            pl.BlockSpec((NC, blkn, OUT), lambda i: (0, i, 0)),
            pl.BlockSpec((blkn, OUT), lambda i: (R, 0)),
            pl.BlockSpec((OUT,), lambda i: (0,)),
        ],
        out_specs=pl.BlockSpec((blkn, OUT), lambda i: (i, 0)),
        out_shape=jax.ShapeDtypeStruct((N, OUT), jnp.float32),
    )


def kernel(x, edge_index, edge_type, coeff, bases, h_bias, loop_weight):
    N, IN = x.shape
    R, B = coeff.shape
    OUT = bases.shape[2]
    E = edge_type.shape[0]
    RN = R * N
    assert E % (NW * CH) == 0 and N % NS == 0 and OUT == 128

    ei_flat = edge_index.reshape(2 * E)
    typ = edge_type.astype(jnp.int32)

    xws = _make_xw(N, IN, OUT, R, B, blkn=N)(coeff, bases, loop_weight, x)
    degp = _make_deg(N, E, RN)(ei_flat, typ, jnp.zeros((RN,), jnp.float32))
    dinv = _make_dinv(RN)(degp)
    accp = _make_scatter(N, E, OUT, RN)(
        ei_flat, typ, xws, dinv, jnp.zeros((N // 10, OUT), jnp.float32))
    h = _make_final(N, OUT, R, blkn=N)(accp, xws, h_bias)
    return h
